# baseline (speedup 1.0000x reference)
import functools

import jax
import jax.numpy as jnp
from jax import lax
from jax.experimental import pallas as pl
from jax.experimental.pallas import tpu as pltpu
from jax.experimental.pallas import tpu_sc as plsc

N_NODES = 10000
N_EDGES = 320000
D_IN = 128
D_HID = 3
D_OUT = 4

_NC, _NS, _L = 2, 16, 16
_NW = _NC * _NS
_NP = 10240
_SLC = _NP // _NS
_EPH = N_EDGES // _NS
_HALF = _NP // _NC
_CH = 4000
_NCH = _EPH // _CH


@functools.cache
def _sc_kernel():
    mesh = plsc.VectorSubcoreMesh(
        core_axis_name="c", subcore_axis_name="s",
        num_cores=_NC, num_subcores=_NS)
    params = pltpu.CompilerParams(
        needs_layout_passes=False, use_tc_tiling_on_sc=False)

    @functools.partial(
        pl.kernel,
        mesh=mesh,
        out_type=[
            jax.ShapeDtypeStruct((_NC * 128, 128), jnp.float32),
            jax.ShapeDtypeStruct((_NC * _NP,), jnp.float32),
        ],
        compiler_params=params,
        scratch_types=[
            pltpu.VMEM((_CH,), jnp.int32),
            pltpu.VMEM((_NP // _L, _L), jnp.float32),
            pltpu.VMEM((_NP // _L,), jnp.int32),
            pltpu.VMEM((128,), jnp.int32),
            pltpu.VMEM((D_HID * _SLC,), jnp.float32),
            pltpu.VMEM((_SLC // _L, _L), jnp.float32),
            pltpu.VMEM((_SLC,), jnp.float32),
            pltpu.VMEM((D_HID * _SLC,), jnp.float32),
            pltpu.VMEM((D_HID * _NP,), jnp.float32),
            pltpu.VMEM((128, 128), jnp.float32),
            pltpu.VMEM_SHARED((_NP // _L, _L), jnp.float32),
            pltpu.VMEM_SHARED((D_HID * _NP,), jnp.float32),
            pltpu.VMEM_SHARED((128, 128), jnp.float32),
        ],
    )
    def sc_mega(xw_hbm, epk_hbm, acc_hbm, deg_hbm,
                epk_v, deg_v, idx_v, idxa_v, xw_v, red_v, dslc_v,
                ysl_v, y_v, acc_v, sh_deg, sh_y, sh_acc):
        sc = lax.axis_index("c")
        t = lax.axis_index("s")

        def edge_chunks(process):
            for c in range(_NCH):
                pltpu.sync_copy(
                    epk_hbm.at[pl.ds(t * _EPH + c * _CH, _CH)], epk_v)
                process(epk_v)

        for ch in range(D_HID):
            pltpu.sync_copy(
                xw_hbm.at[pl.ds(ch * _NP + t * _SLC, _SLC)],
                xw_v.at[pl.ds(ch * _SLC, _SLC)])

        zeros = jnp.zeros((_L,), jnp.float32)
        iota = lax.iota(jnp.int32, _L)

        @plsc.parallel_loop(0, _SLC // _L, unroll=4)
        def zred(i):
            red_v[i, :] = zeros
            idx_v[pl.ds(i * _L, _L)] = iota + i * _L

        @plsc.parallel_loop(0, 128 // _L, unroll=4)
        def mkidxa(i):
            idxa_v[pl.ds(i * _L, _L)] = iota + i * _L

        pltpu.sync_copy(red_v, sh_deg.at[pl.ds(t * (_SLC // _L), _SLC // _L)])

        @plsc.parallel_loop(0, _NP // _L, unroll=8)
        def zdeg(i):
            deg_v[i, :] = zeros

        ones = jnp.ones((_L,), jnp.float32)
        lanemask = jnp.full((_L,), _L - 1, jnp.int32)

        def hist_chunk(buf):
            @plsc.parallel_loop(0, _CH // _L, unroll=8)
            def hist(g):
                d16 = lax.shift_right_logical(buf[pl.ds(g * _L, _L)], 14)
                plsc.addupdate_scatter(
                    deg_v,
                    [lax.shift_right_logical(d16, 4), d16 & lanemask],
                    ones)

        edge_chunks(hist_chunk)

        plsc.subcore_barrier()

        pltpu.sync_copy(deg_v, sh_deg.at[idx_v], add=True)

        plsc.subcore_barrier()

        pltpu.sync_copy(sh_deg.at[pl.ds(t * (_SLC // _L), _SLC // _L)], red_v)

        half = jnp.full((_L,), 0.5, jnp.float32)
        three_half = jnp.full((_L,), 1.5, jnp.float32)
        magic = jnp.full((_L,), 0x5F3759DF, jnp.int32)

        @plsc.parallel_loop(0, _SLC // _L, unroll=2)
        def newton(g):
            sl = pl.ds(g * _L, _L)
            d16 = red_v[g, :] + 1.0
            dslc_v[sl] = d16
            i32 = magic - lax.shift_right_arithmetic(
                plsc.bitcast(d16, jnp.int32), 1)
            g0 = plsc.bitcast(i32, jnp.float32)
            hx = d16 * half
            for _ in range(3):
                g0 = g0 * (three_half - hx * g0 * g0)
            for ch in range(D_HID):
                csl = pl.ds(ch * _SLC + g * _L, _L)
                ysl_v[csl] = xw_v[csl] * g0

        pltpu.sync_copy(dslc_v, deg_hbm.at[pl.ds(sc * _NP + t * _SLC, _SLC)])
        for ch in range(D_HID):
            pltpu.sync_copy(
                ysl_v.at[pl.ds(ch * _SLC, _SLC)],
                sh_y.at[pl.ds(ch * _NP + t * _SLC, _SLC)])

        plsc.subcore_barrier()

        pltpu.sync_copy(sh_y, y_v)

        @plsc.parallel_loop(0, 128, unroll=2)
        def zacc(i):
            for c in range(8):
                acc_v[i, pl.ds(c * _L, _L)] = zeros

        srcmask = jnp.full((_L,), (1 << 14) - 1, jnp.int32)
        colmask = jnp.full((_L,), 127, jnp.int32)
        lo = sc * _HALF

        def edge_chunk(buf):
            @plsc.parallel_loop(0, _CH // _L, unroll=4)
            def edge(g):
                e16 = buf[pl.ds(g * _L, _L)]
                s16 = e16 & srcmask
                d16 = lax.shift_right_logical(e16, 14) - lo
                m = (d16 >= 0) & (d16 < _HALF)
                dm = jnp.where(m, d16, 0)
                for ch in range(D_HID):
                    v = plsc.load_gather(y_v, [s16 + jnp.int32(ch * _NP)])
                    full = dm + jnp.int32(ch * _HALF)
                    plsc.addupdate_scatter(
                        acc_v,
                        [lax.shift_right_logical(full, 7), full & colmask],
                        v, mask=m)

        edge_chunks(edge_chunk)

        pltpu.sync_copy(acc_v.at[pl.ds(120, 8)], sh_acc.at[pl.ds(t * 8, 8)])
        plsc.subcore_barrier()
        pltpu.sync_copy(acc_v, sh_acc.at[idxa_v], add=True)
        plsc.subcore_barrier()
        pltpu.sync_copy(sh_acc.at[pl.ds(t * 8, 8)], acc_v.at[pl.ds(120, 8)])
        pltpu.sync_copy(acc_v.at[pl.ds(120, 8)],
                        acc_hbm.at[pl.ds(sc * 128 + t * 8, 8)])

    return sc_mega


def _pre_body(x_ref, w_ref, y_ref):
    xw_t = lax.dot_general(
        w_ref[...], x_ref[...], (((0,), (1,)), ((), ())),
        preferred_element_type=jnp.float32)
    y_ref[:, :N_NODES] = xw_t
    y_ref[:, N_NODES:] = jnp.zeros((D_HID, _NP - N_NODES), jnp.float32)


def _fin_body(accp_ref, deg_ref, xw_ref, b_ref, w2_ref, b2_ref, h_ref, z_ref):
    acc = jnp.concatenate([accp_ref[0], accp_ref[1]], axis=1)
    dinv = lax.rsqrt(deg_ref[0:1, :])
    y = xw_ref[...] * dinv
    h = jnp.maximum(dinv * (acc + y) + b_ref[...], 0.0)
    h_ref[...] = h
    z_ref[...] = lax.dot_general(
        w2_ref[...], h, (((0,), (0,)), ((), ())),
        preferred_element_type=jnp.float32) + b2_ref[...]


_pre_call = pl.pallas_call(
    _pre_body,
    out_shape=jax.ShapeDtypeStruct((D_HID, _NP), jnp.float32),
)

_fin_call = pl.pallas_call(
    _fin_body,
    out_shape=[
        jax.ShapeDtypeStruct((D_HID, _NP), jnp.float32),
        jax.ShapeDtypeStruct((D_OUT, _NP), jnp.float32),
    ],
)


def kernel(x, edges, W, b, W2, b2):
    src = edges[0].astype(jnp.int32)
    dst = edges[1].astype(jnp.int32)
    packed = src | (dst << jnp.int32(14))

    xw_t = _pre_call(x, W)
    sc_mega = _sc_kernel()
    accp, deg = sc_mega(xw_t.reshape(D_HID * _NP), packed)
    acc3 = accp.reshape(_NC, 128 * 128)[:, :D_HID * _HALF]
    h_t, z_t = _fin_call(
        acc3.reshape(_NC, D_HID, _HALF), deg.reshape(_NC, _NP), xw_t,
        b.reshape(D_HID, 1), W2, b2.reshape(D_OUT, 1))
    return h_t[:, :N_NODES].T, z_t[:, :N_NODES].T


# speedup vs baseline: 1.0724x; 1.0724x over previous
import functools

import jax
import jax.numpy as jnp
from jax import lax
from jax.experimental import pallas as pl
from jax.experimental.pallas import tpu as pltpu
from jax.experimental.pallas import tpu_sc as plsc

N_NODES = 10000
N_EDGES = 320000
D_IN = 128
D_HID = 3
D_OUT = 4

_NC, _NS, _L = 2, 16, 16
_NW = _NC * _NS
_NP = 10240
_SLC = _NP // _NS
_EPH = N_EDGES // _NS
_HALF = _NP // _NC
_CH = 4000
_NCH = _EPH // _CH


@functools.cache
def _sc_kernel():
    mesh = plsc.VectorSubcoreMesh(
        core_axis_name="c", subcore_axis_name="s",
        num_cores=_NC, num_subcores=_NS)
    params = pltpu.CompilerParams(
        needs_layout_passes=False, use_tc_tiling_on_sc=False)

    @functools.partial(
        pl.kernel,
        mesh=mesh,
        out_type=[
            jax.ShapeDtypeStruct((_NC * 128, 128), jnp.float32),
            jax.ShapeDtypeStruct((_NC * _NP,), jnp.float32),
        ],
        compiler_params=params,
        scratch_types=[
            pltpu.VMEM((_CH,), jnp.int32),
            pltpu.VMEM((_CH,), jnp.int32),
            pltpu.VMEM((_NP // _L, _L), jnp.float32),
            pltpu.VMEM((_NP // _L,), jnp.int32),
            pltpu.VMEM((128,), jnp.int32),
            pltpu.VMEM((D_HID * _SLC,), jnp.float32),
            pltpu.VMEM((_SLC // _L, _L), jnp.float32),
            pltpu.VMEM((_SLC,), jnp.float32),
            pltpu.VMEM((D_HID * _SLC,), jnp.float32),
            pltpu.VMEM((D_HID * _NP,), jnp.float32),
            pltpu.VMEM((128, 128), jnp.float32),
            pltpu.VMEM_SHARED((_NP // _L, _L), jnp.float32),
            pltpu.VMEM_SHARED((D_HID * _NP,), jnp.float32),
            pltpu.VMEM_SHARED((128, 128), jnp.float32),
            pltpu.SemaphoreType.DMA,
            pltpu.SemaphoreType.DMA,
        ],
    )
    def sc_mega(xw_hbm, epk_hbm, acc_hbm, deg_hbm,
                epk0_v, epk1_v, deg_v, idx_v, idxa_v, xw_v, red_v, dslc_v,
                ysl_v, y_v, acc_v, sh_deg, sh_y, sh_acc, sem0, sem1):
        sc = lax.axis_index("c")
        t = lax.axis_index("s")

        bufs = (epk0_v, epk1_v)
        sems = (sem0, sem1)

        def edge_chunks(process):
            copies = [None] * _NCH
            copies[0] = pltpu.async_copy(
                epk_hbm.at[pl.ds(t * _EPH, _CH)], bufs[0], sems[0])
            for c in range(_NCH):
                if c + 1 < _NCH:
                    copies[c + 1] = pltpu.async_copy(
                        epk_hbm.at[pl.ds(t * _EPH + (c + 1) * _CH, _CH)],
                        bufs[(c + 1) % 2], sems[(c + 1) % 2])
                copies[c].wait()
                process(bufs[c % 2])

        for ch in range(D_HID):
            pltpu.sync_copy(
                xw_hbm.at[pl.ds(ch * _NP + t * _SLC, _SLC)],
                xw_v.at[pl.ds(ch * _SLC, _SLC)])

        zeros = jnp.zeros((_L,), jnp.float32)
        iota = lax.iota(jnp.int32, _L)

        @plsc.parallel_loop(0, _SLC // _L, unroll=4)
        def zred(i):
            red_v[i, :] = zeros
            idx_v[pl.ds(i * _L, _L)] = iota + i * _L

        @plsc.parallel_loop(0, 128 // _L, unroll=4)
        def mkidxa(i):
            idxa_v[pl.ds(i * _L, _L)] = iota + i * _L

        pltpu.sync_copy(red_v, sh_deg.at[pl.ds(t * (_SLC // _L), _SLC // _L)])

        @plsc.parallel_loop(0, _NP // _L, unroll=8)
        def zdeg(i):
            deg_v[i, :] = zeros

        ones = jnp.ones((_L,), jnp.float32)
        lanemask = jnp.full((_L,), _L - 1, jnp.int32)

        def hist_chunk(buf):
            @plsc.parallel_loop(0, _CH // _L, unroll=8)
            def hist(g):
                d16 = lax.shift_right_logical(buf[pl.ds(g * _L, _L)], 14)
                plsc.addupdate_scatter(
                    deg_v,
                    [lax.shift_right_logical(d16, 4), d16 & lanemask],
                    ones)

        edge_chunks(hist_chunk)

        plsc.subcore_barrier()

        pltpu.sync_copy(deg_v, sh_deg.at[idx_v], add=True)

        plsc.subcore_barrier()

        pltpu.sync_copy(sh_deg.at[pl.ds(t * (_SLC // _L), _SLC // _L)], red_v)

        half = jnp.full((_L,), 0.5, jnp.float32)
        three_half = jnp.full((_L,), 1.5, jnp.float32)
        magic = jnp.full((_L,), 0x5F3759DF, jnp.int32)

        @plsc.parallel_loop(0, _SLC // _L, unroll=2)
        def newton(g):
            sl = pl.ds(g * _L, _L)
            d16 = red_v[g, :] + 1.0
            dslc_v[sl] = d16
            i32 = magic - lax.shift_right_arithmetic(
                plsc.bitcast(d16, jnp.int32), 1)
            g0 = plsc.bitcast(i32, jnp.float32)
            hx = d16 * half
            for _ in range(3):
                g0 = g0 * (three_half - hx * g0 * g0)
            for ch in range(D_HID):
                csl = pl.ds(ch * _SLC + g * _L, _L)
                ysl_v[csl] = xw_v[csl] * g0

        pltpu.sync_copy(dslc_v, deg_hbm.at[pl.ds(sc * _NP + t * _SLC, _SLC)])
        for ch in range(D_HID):
            pltpu.sync_copy(
                ysl_v.at[pl.ds(ch * _SLC, _SLC)],
                sh_y.at[pl.ds(ch * _NP + t * _SLC, _SLC)])

        plsc.subcore_barrier()

        pltpu.sync_copy(sh_y, y_v)

        @plsc.parallel_loop(0, 128, unroll=2)
        def zacc(i):
            for c in range(8):
                acc_v[i, pl.ds(c * _L, _L)] = zeros

        srcmask = jnp.full((_L,), (1 << 14) - 1, jnp.int32)
        colmask = jnp.full((_L,), 127, jnp.int32)
        lo = sc * _HALF

        def edge_chunk(buf):
            @plsc.parallel_loop(0, _CH // _L, unroll=8)
            def edge(g):
                e16 = buf[pl.ds(g * _L, _L)]
                s16 = e16 & srcmask
                d16 = lax.shift_right_logical(e16, 14) - lo
                m = (d16 >= 0) & (d16 < _HALF)
                dm = jnp.where(m, d16, 0)
                for ch in range(D_HID):
                    v = plsc.load_gather(y_v, [s16 + jnp.int32(ch * _NP)])
                    full = dm + jnp.int32(ch * _HALF)
                    plsc.addupdate_scatter(
                        acc_v,
                        [lax.shift_right_logical(full, 7), full & colmask],
                        v, mask=m)

        edge_chunks(edge_chunk)

        pltpu.sync_copy(acc_v.at[pl.ds(120, 8)], sh_acc.at[pl.ds(t * 8, 8)])
        plsc.subcore_barrier()
        pltpu.sync_copy(acc_v, sh_acc.at[idxa_v], add=True)
        plsc.subcore_barrier()
        pltpu.sync_copy(sh_acc.at[pl.ds(t * 8, 8)], acc_v.at[pl.ds(120, 8)])
        pltpu.sync_copy(acc_v.at[pl.ds(120, 8)],
                        acc_hbm.at[pl.ds(sc * 128 + t * 8, 8)])

    return sc_mega


def _pre_body(x_ref, w_ref, y_ref):
    xw_t = lax.dot_general(
        w_ref[...], x_ref[...], (((0,), (1,)), ((), ())),
        preferred_element_type=jnp.float32)
    y_ref[:, :N_NODES] = xw_t
    y_ref[:, N_NODES:] = jnp.zeros((D_HID, _NP - N_NODES), jnp.float32)


def _fin_body(accp_ref, deg_ref, xw_ref, b_ref, w2_ref, b2_ref, h_ref, z_ref):
    acc = jnp.concatenate([accp_ref[0], accp_ref[1]], axis=1)
    dinv = lax.rsqrt(deg_ref[0:1, :])
    y = xw_ref[...] * dinv
    h = jnp.maximum(dinv * (acc + y) + b_ref[...], 0.0)
    h_ref[...] = h
    z_ref[...] = lax.dot_general(
        w2_ref[...], h, (((0,), (0,)), ((), ())),
        preferred_element_type=jnp.float32) + b2_ref[...]


_pre_call = pl.pallas_call(
    _pre_body,
    out_shape=jax.ShapeDtypeStruct((D_HID, _NP), jnp.float32),
)

_fin_call = pl.pallas_call(
    _fin_body,
    out_shape=[
        jax.ShapeDtypeStruct((D_HID, _NP), jnp.float32),
        jax.ShapeDtypeStruct((D_OUT, _NP), jnp.float32),
    ],
)


def kernel(x, edges, W, b, W2, b2):
    src = edges[0].astype(jnp.int32)
    dst = edges[1].astype(jnp.int32)
    packed = src | (dst << jnp.int32(14))

    xw_t = _pre_call(x, W)
    sc_mega = _sc_kernel()
    accp, deg = sc_mega(xw_t.reshape(D_HID * _NP), packed)
    acc3 = accp.reshape(_NC, 128 * 128)[:, :D_HID * _HALF]
    h_t, z_t = _fin_call(
        acc3.reshape(_NC, D_HID, _HALF), deg.reshape(_NC, _NP), xw_t,
        b.reshape(D_HID, 1), W2, b2.reshape(D_OUT, 1))
    return h_t[:, :N_NODES].T, z_t[:, :N_NODES].T
